# baseline (speedup 1.0000x reference)
import functools

import jax
import jax.numpy as jnp
from jax import lax
from jax.experimental import pallas as pl
from jax.experimental.pallas import tpu as pltpu
from jax.experimental.pallas import tpu_sc as plsc

L = 16
NC = 2
NS = 16
NW = NC * NS
IDX_CHUNK = 128


@functools.lru_cache(maxsize=None)
def _build(B, D, dtype_name):
    dtype = jnp.dtype(dtype_name)
    bpw = B // NW
    n_chunks = bpw // IDX_CHUNK
    mesh = plsc.VectorSubcoreMesh(core_axis_name="c", subcore_axis_name="s")

    @functools.partial(
        pl.kernel,
        mesh=mesh,
        compiler_params=pltpu.CompilerParams(
            use_tc_tiling_on_sc=False, needs_layout_passes=False),
        out_type=jax.ShapeDtypeStruct((B, 7 * D), dtype),
        scratch_types=(
            [pltpu.VMEM((bpw,), jnp.int32) for _ in range(5)]
            + [pltpu.VMEM((bpw, D), dtype) for _ in range(5)]
            + [
                pltpu.VMEM((bpw,), dtype),
                pltpu.VMEM((bpw,), dtype),
                pltpu.VMEM((D,), dtype),
                pltpu.VMEM((D,), dtype),
                pltpu.VMEM((D,), dtype),
                pltpu.VMEM((D,), dtype),
                pltpu.VMEM((bpw, D), dtype),
                pltpu.VMEM((bpw, D), dtype),
                pltpu.SemaphoreType.DMA,
                pltpu.SemaphoreType.DMA,
            ]
        ),
    )
    def embed(uid_h, bid_h, lid_h, aid_h, pid_h, age_h, year_h,
              ut_h, bt_h, lt_h, at_h, pt_h,
              wa_h, ba_h, wy_h, by_h,
              out_h,
              iu, ib, il, ia, ip,
              ru, rb, rl, ra, rp,
              age_v, year_v, wa_v, ba_v, wy_v, by_v,
              aband, yband, gsem, osem):
        wid = lax.axis_index("s") * NC + lax.axis_index("c")
        base = wid * bpw

        for src, dst in ((uid_h, iu), (bid_h, ib), (lid_h, il),
                         (aid_h, ia), (pid_h, ip)):
            pltpu.sync_copy(src.at[pl.ds(base, bpw)], dst)

        gathers = []
        for tab, idx, rows in ((ut_h, iu, ru), (bt_h, ib, rb),
                               (lt_h, il, rl), (at_h, ia, ra),
                               (pt_h, ip, rp)):
            for j in range(n_chunks):
                sl = pl.ds(j * IDX_CHUNK, IDX_CHUNK)
                gathers.append(
                    pltpu.async_copy(tab.at[idx.at[sl]], rows.at[sl], gsem))

        pltpu.sync_copy(age_h.at[pl.ds(base, bpw)], age_v)
        pltpu.sync_copy(year_h.at[pl.ds(base, bpw)], year_v)
        pltpu.sync_copy(wa_h, wa_v)
        pltpu.sync_copy(ba_h, ba_v)
        pltpu.sync_copy(wy_h, wy_v)
        pltpu.sync_copy(by_h, by_v)

        wa_lo = wa_v[pl.ds(0, L)]
        wa_hi = wa_v[pl.ds(L, L)]
        ba_lo = ba_v[pl.ds(0, L)]
        ba_hi = ba_v[pl.ds(L, L)]
        wy_lo = wy_v[pl.ds(0, L)]
        wy_hi = wy_v[pl.ds(L, L)]
        by_lo = by_v[pl.ds(0, L)]
        by_hi = by_v[pl.ds(L, L)]

        def dense_row(b, carry):
            lane = jnp.full((L,), b, jnp.int32)
            av = plsc.load_gather(age_v, [lane])
            yv = plsc.load_gather(year_v, [lane])
            aband[b, pl.ds(0, L)] = av * wa_lo + ba_lo
            aband[b, pl.ds(L, L)] = av * wa_hi + ba_hi
            yband[b, pl.ds(0, L)] = yv * wy_lo + by_lo
            yband[b, pl.ds(L, L)] = yv * wy_hi + by_hi
            return carry

        lax.fori_loop(0, bpw, dense_row, 0)

        for h in gathers:
            h.wait()

        outs = []
        for rows, off in ((ru, 0), (rb, D), (rl, 2 * D), (aband, 3 * D),
                          (ra, 4 * D), (yband, 5 * D), (rp, 6 * D)):
            outs.append(
                pltpu.async_copy(
                    rows, out_h.at[pl.ds(base, bpw), pl.ds(off, D)], osem))
        for h in outs:
            h.wait()

    return embed


def kernel(user_id, book_id, location_id, age, author_id, year, publisher_id,
           user_table, book_table, location_table, author_table,
           publisher_table, w_age, b_age, w_year, b_year):
    B = user_id.shape[0]
    D = user_table.shape[1]
    embed = _build(B, D, str(user_table.dtype))
    return embed(user_id, book_id, location_id, author_id, publisher_id,
                 age.reshape(B), year.reshape(B),
                 user_table, book_table, location_table, author_table,
                 publisher_table,
                 w_age.reshape(D), b_age.reshape(D),
                 w_year.reshape(D), b_year.reshape(D))
